# Initial kernel scaffold; baseline (speedup 1.0000x reference)
import jax
import jax.numpy as jnp
from jax.experimental import pallas as pl


def kernel(x):
    raise NotImplementedError("write your pallas kernel here")


# speedup vs baseline: 3.2005x; 3.2005x over previous
import functools

import jax
import jax.numpy as jnp
import numpy as np
from jax import lax
from jax.experimental import pallas as pl
from jax.experimental.pallas import tpu as pltpu
from jax.experimental.pallas import tpu_sc as plsc

NC = 2
NS = 16
LANES = 16
NW = NC * NS

N = 2 * 4096 * 4096
SHARD = N // NW

CHUNK_A = 65536
CHUNK_B = 32768
WIN = 128

MAGICF = np.float32(12582912.0)
MAGICI = np.int32(0x4B400000)

_mesh = plsc.VectorSubcoreMesh(core_axis_name="c", subcore_axis_name="s")


def _wid():
    return lax.axis_index("s") * NC + lax.axis_index("c")


@functools.partial(
    pl.kernel,
    out_type=(
        jax.ShapeDtypeStruct((NW, LANES), jnp.float32),
        jax.ShapeDtypeStruct((NW, LANES), jnp.float32),
    ),
    mesh=_mesh,
    compiler_params=pltpu.CompilerParams(needs_layout_passes=False),
    scratch_types=[
        pltpu.VMEM((CHUNK_A,), jnp.float32),
        pltpu.VMEM((LANES,), jnp.float32),
        pltpu.VMEM((LANES,), jnp.float32),
    ],
)
def _minmax_kernel(x_hbm, min_hbm, max_hbm, buf, mn_buf, mx_buf):
    base = _wid() * SHARD

    def chunk(c, carry):
        pltpu.sync_copy(x_hbm.at[pl.ds(base + c * CHUNK_A, CHUNK_A)], buf)

        def vred(i, carry2):
            mn, mx = carry2
            v = buf[pl.ds(i * LANES, LANES)]
            return jnp.minimum(mn, v), jnp.maximum(mx, v)

        return lax.fori_loop(0, CHUNK_A // LANES, vred, carry)

    init = (jnp.full((LANES,), jnp.inf, jnp.float32),
            jnp.full((LANES,), -jnp.inf, jnp.float32))
    mn, mx = lax.fori_loop(0, SHARD // CHUNK_A, chunk, init)
    mn_buf[...] = mn
    mx_buf[...] = mx
    wid = _wid()
    pltpu.sync_copy(mn_buf, min_hbm.at[wid])
    pltpu.sync_copy(mx_buf, max_hbm.at[wid])


def _ce(a, b):
    return jnp.maximum(a, b), jnp.minimum(a, b)


def _windows(inb, outb, sv, iv, nwin):
    idx8 = lax.iota(jnp.int32, LANES) * 8

    def window(w, _):
        base = w * WIN
        idxs = [idx8 + (base + j) for j in range(8)]
        qs = []
        for j in range(8):
            xv = plsc.load_gather(inb, [idxs[j]])
            t = xv * iv
            u = t + MAGICF
            q0 = plsc.bitcast(u, jnp.int32) - MAGICI
            qs.append(jnp.minimum(jnp.maximum(q0, -128), 127))
        ms = [jnp.abs(q) for q in qs]
        a0, a1 = _ce(ms[0], ms[1])
        a2, a3 = _ce(ms[2], ms[3])
        a0, a2 = _ce(a0, a2)
        a1, a3 = _ce(a1, a3)
        a1, a2 = _ce(a1, a2)
        b0, b1 = _ce(ms[4], ms[5])
        b2, b3 = _ce(ms[6], ms[7])
        b0, b2 = _ce(b0, b2)
        b1, b3 = _ce(b1, b3)
        b1, b2 = _ce(b1, b2)
        thr = jnp.maximum(
            jnp.maximum(b3, jnp.minimum(a0, b2)),
            jnp.maximum(jnp.minimum(a1, b1),
                        jnp.maximum(jnp.minimum(a2, b0), a3)))
        for j in range(8):
            hi = ms[j] & -16
            low = ms[j] - hi
            keep = jnp.where(ms[j] >= thr, low, 0)
            k2 = hi + keep
            qp = jnp.where(qs[j] < 0, -k2, k2)
            plsc.store_scatter(outb, [idxs[j]], qp.astype(jnp.float32) * sv)
        return 0

    lax.fori_loop(0, nwin, window, 0)


@functools.partial(
    pl.kernel,
    out_type=jax.ShapeDtypeStruct((N,), jnp.float32),
    mesh=_mesh,
    compiler_params=pltpu.CompilerParams(needs_layout_passes=False),
    scratch_types=[
        pltpu.VMEM((CHUNK_B,), jnp.float32),
        pltpu.VMEM((CHUNK_B,), jnp.float32),
        pltpu.VMEM((LANES,), jnp.float32),
        pltpu.VMEM((LANES,), jnp.float32),
    ],
)
def _quant_kernel(x_hbm, scale_hbm, inv_hbm, out_hbm, inb, outb, sc_buf, iv_buf):
    base = _wid() * SHARD
    pltpu.sync_copy(scale_hbm, sc_buf)
    pltpu.sync_copy(inv_hbm, iv_buf)
    sv = sc_buf[...]
    iv = iv_buf[...]

    def chunk(c, _):
        off = base + c * CHUNK_B
        pltpu.sync_copy(x_hbm.at[pl.ds(off, CHUNK_B)], inb)
        _windows(inb, outb, sv, iv, CHUNK_B // WIN)
        pltpu.sync_copy(outb, out_hbm.at[pl.ds(off, CHUNK_B)])
        return 0

    lax.fori_loop(0, SHARD // CHUNK_B, chunk, 0)


def kernel(x):
    xf = x.reshape(-1)
    mn, mx = _minmax_kernel(xf)
    r_min = jnp.minimum(jnp.min(mn), 0.0)
    r_max = jnp.maximum(jnp.max(mx), 0.0)
    scale = jnp.maximum((r_max - r_min) / 255, jnp.float32(1e-8))
    inv = jnp.float32(1.0) / scale
    scale_vec = jnp.full((LANES,), scale, jnp.float32)
    inv_vec = jnp.full((LANES,), inv, jnp.float32)
    out = _quant_kernel(xf, scale_vec, inv_vec)
    return out.reshape(x.shape)


# speedup vs baseline: 5.3517x; 1.6722x over previous
import functools

import jax
import jax.numpy as jnp
import numpy as np
from jax import lax
from jax.experimental import pallas as pl
from jax.experimental.pallas import tpu as pltpu
from jax.experimental.pallas import tpu_sc as plsc

NC = 2
NS = 16
LANES = 16
NW = NC * NS

N = 2 * 4096 * 4096
SHARD = N // NW

CHUNK_A = 32768
CHUNK_B = 16384
NCHUNK_A = SHARD // CHUNK_A
NCHUNK_B = SHARD // CHUNK_B
WIN = 128

MAGICF = np.float32(12582912.0)
MAGICI = np.int32(0x4B400000)
SIGNBIT = np.int32(np.uint32(0x80000000).view(np.int32))

_mesh = plsc.VectorSubcoreMesh(core_axis_name="c", subcore_axis_name="s")
_params = pltpu.CompilerParams(needs_layout_passes=False)


def _wid():
    return lax.axis_index("s") * NC + lax.axis_index("c")


@functools.partial(
    pl.kernel,
    out_type=(
        jax.ShapeDtypeStruct((NW, LANES), jnp.float32),
        jax.ShapeDtypeStruct((NW, LANES), jnp.float32),
    ),
    mesh=_mesh,
    compiler_params=_params,
    scratch_types=[
        pltpu.VMEM((CHUNK_A,), jnp.float32),
        pltpu.VMEM((CHUNK_A,), jnp.float32),
        pltpu.VMEM((LANES,), jnp.float32),
        pltpu.VMEM((LANES,), jnp.float32),
        pltpu.SemaphoreType.DMA,
        pltpu.SemaphoreType.DMA,
    ],
)
def _minmax_kernel(x_hbm, min_hbm, max_hbm, in0, in1, mn_buf, mx_buf, si0, si1):
    wid = _wid()
    base = wid * SHARD
    bufs = ((in0, si0), (in1, si1))

    def src(c):
        return x_hbm.at[pl.ds(base + c * CHUNK_A, CHUNK_A)]

    pltpu.async_copy(src(0), in0, si0)
    pltpu.async_copy(src(1), in1, si1)

    def pair(p, carry):
        for b, (inb, si) in enumerate(bufs):
            c = p * 2 + b
            pltpu.make_async_copy(src(c), inb, si).wait()

            def vred(i, acc):
                off = i * 128
                return tuple(
                    jnp.minimum(acc[k], inb[pl.ds(off + k * LANES, LANES)])
                    if k < 8 else
                    jnp.maximum(acc[k], inb[pl.ds(off + (k - 8) * LANES, LANES)])
                    for k in range(16)
                )

            carry = lax.fori_loop(0, CHUNK_A // 128, vred, carry)

            @pl.when(c + 2 < NCHUNK_A)
            def _():
                pltpu.async_copy(src(c + 2), inb, si)
        return carry

    init = tuple(
        jnp.full((LANES,), jnp.inf if k < 8 else -jnp.inf, jnp.float32)
        for k in range(16)
    )
    acc = lax.fori_loop(0, NCHUNK_A // 2, pair, init)
    mn = acc[0]
    mx = acc[8]
    for k in range(1, 8):
        mn = jnp.minimum(mn, acc[k])
        mx = jnp.maximum(mx, acc[8 + k])
    mn_buf[...] = mn
    mx_buf[...] = mx
    pltpu.sync_copy(mn_buf, min_hbm.at[wid])
    pltpu.sync_copy(mx_buf, max_hbm.at[wid])


def _ce(a, b):
    return jnp.maximum(a, b), jnp.minimum(a, b)


def _windows(inb, outb, sv, iv, nwin):
    idx8 = lax.iota(jnp.int32, LANES) * 8
    zero = jnp.zeros((LANES,), jnp.int32)

    @plsc.parallel_loop(0, nwin, 1, unroll=2)
    def window(w):
        base = w * WIN
        idxs = [idx8 + (base + j) for j in range(8)]
        xs = [plsc.load_gather(inb, [idxs[j]]) for j in range(8)]
        ms = []
        for j in range(8):
            t = xs[j] * iv
            u = t + MAGICF
            q0 = plsc.bitcast(u, jnp.int32) - MAGICI
            ms.append(jnp.abs(jnp.minimum(jnp.maximum(q0, -128), 127)))
        a0, a1 = _ce(ms[0], ms[1])
        a2, a3 = _ce(ms[2], ms[3])
        a0, a2 = _ce(a0, a2)
        a1, a3 = _ce(a1, a3)
        a1, a2 = _ce(a1, a2)
        b0, b1 = _ce(ms[4], ms[5])
        b2, b3 = _ce(ms[6], ms[7])
        b0, b2 = _ce(b0, b2)
        b1, b3 = _ce(b1, b3)
        b1, b2 = _ce(b1, b2)
        thr = jnp.maximum(
            jnp.maximum(b3, jnp.minimum(a0, b2)),
            jnp.maximum(jnp.minimum(a1, b1),
                        jnp.maximum(jnp.minimum(a2, b0), a3)))
        for j in range(8):
            low = ms[j] & 15
            drop = jnp.where(ms[j] >= thr, zero, low)
            k2 = ms[j] - drop
            fs = k2.astype(jnp.float32) * sv
            ob = plsc.bitcast(
                plsc.bitcast(fs, jnp.int32)
                | (plsc.bitcast(xs[j], jnp.int32) & SIGNBIT),
                jnp.float32)
            plsc.store_scatter(outb, [idxs[j]], ob)


@functools.partial(
    pl.kernel,
    out_type=jax.ShapeDtypeStruct((N,), jnp.float32),
    mesh=_mesh,
    compiler_params=_params,
    scratch_types=[
        pltpu.VMEM((CHUNK_B,), jnp.float32),
        pltpu.VMEM((CHUNK_B,), jnp.float32),
        pltpu.VMEM((CHUNK_B,), jnp.float32),
        pltpu.VMEM((CHUNK_B,), jnp.float32),
        pltpu.VMEM((LANES,), jnp.float32),
        pltpu.VMEM((LANES,), jnp.float32),
        pltpu.SemaphoreType.DMA,
        pltpu.SemaphoreType.DMA,
        pltpu.SemaphoreType.DMA,
        pltpu.SemaphoreType.DMA,
    ],
)
def _quant_kernel(x_hbm, scale_hbm, inv_hbm, out_hbm,
                  in0, in1, out0, out1, sc_buf, iv_buf, si0, si1, so0, so1):
    base = _wid() * SHARD
    pltpu.sync_copy(scale_hbm, sc_buf)
    pltpu.sync_copy(inv_hbm, iv_buf)
    sv = sc_buf[...]
    iv = iv_buf[...]
    bufs = ((in0, out0, si0, so0), (in1, out1, si1, so1))

    def src(c):
        return x_hbm.at[pl.ds(base + c * CHUNK_B, CHUNK_B)]

    def dst(c):
        return out_hbm.at[pl.ds(base + c * CHUNK_B, CHUNK_B)]

    pltpu.async_copy(src(0), in0, si0)
    pltpu.async_copy(src(1), in1, si1)

    def pair(p, _):
        for b, (inb, outb, si, so) in enumerate(bufs):
            c = p * 2 + b
            pltpu.make_async_copy(src(c), inb, si).wait()

            @pl.when(c >= 2)
            def _():
                pltpu.make_async_copy(outb, dst(c), so).wait()

            _windows(inb, outb, sv, iv, CHUNK_B // WIN)
            pltpu.async_copy(outb, dst(c), so)

            @pl.when(c + 2 < NCHUNK_B)
            def _():
                pltpu.async_copy(src(c + 2), inb, si)
        return 0

    lax.fori_loop(0, NCHUNK_B // 2, pair, 0)
    pltpu.make_async_copy(out0, dst(0), so0).wait()
    pltpu.make_async_copy(out1, dst(1), so1).wait()


def kernel(x):
    xf = x.reshape(-1)
    mn, mx = _minmax_kernel(xf)
    r_min = jnp.minimum(jnp.min(mn), 0.0)
    r_max = jnp.maximum(jnp.max(mx), 0.0)
    scale = jnp.maximum((r_max - r_min) / 255, jnp.float32(1e-8))
    inv = jnp.float32(1.0) / scale
    scale_vec = jnp.full((LANES,), scale, jnp.float32)
    inv_vec = jnp.full((LANES,), inv, jnp.float32)
    out = _quant_kernel(xf, scale_vec, inv_vec)
    return out.reshape(x.shape)


# speedup vs baseline: 6.8835x; 1.2862x over previous
import functools

import jax
import jax.numpy as jnp
import numpy as np
from jax import lax
from jax.experimental import pallas as pl
from jax.experimental.pallas import tpu as pltpu
from jax.experimental.pallas import tpu_sc as plsc

NC = 2
NS = 16
LANES = 16
NW = NC * NS

R = 8192
C = 4096
ROWS_W = R // NW
RCHUNK = 8
NCHUNK = ROWS_W // RCHUNK
CHUNK = RCHUNK * C
NWIN = CHUNK // 128

MAGICF = np.float32(12582912.0)
MAGICI = np.int32(0x4B400000)
SIGNBIT = np.int32(np.uint32(0x80000000).view(np.int32))

_mesh = plsc.VectorSubcoreMesh(core_axis_name="c", subcore_axis_name="s")
_params = pltpu.CompilerParams(needs_layout_passes=False)


def _wid():
    return lax.axis_index("s") * NC + lax.axis_index("c")


def _rows_in(x_hbm, rr, buf, sem):
    for s in range(RCHUNK):
        pltpu.async_copy(x_hbm.at[rr + s, :], buf.at[pl.ds(s * C, C)], sem)


def _rows_in_wait(x_hbm, rr, buf, sem):
    for s in range(RCHUNK):
        pltpu.make_async_copy(x_hbm.at[rr + s, :], buf.at[pl.ds(s * C, C)],
                              sem).wait()


def _rows_out(out_hbm, rr, buf, sem):
    for s in range(RCHUNK):
        pltpu.async_copy(buf.at[pl.ds(s * C, C)], out_hbm.at[rr + s, :], sem)


def _rows_out_wait(out_hbm, rr, buf, sem):
    for s in range(RCHUNK):
        pltpu.make_async_copy(buf.at[pl.ds(s * C, C)], out_hbm.at[rr + s, :],
                              sem).wait()


@functools.partial(
    pl.kernel,
    out_type=(
        jax.ShapeDtypeStruct((NW * LANES,), jnp.float32),
        jax.ShapeDtypeStruct((NW * LANES,), jnp.float32),
    ),
    mesh=_mesh,
    compiler_params=_params,
    scratch_types=[
        pltpu.VMEM((CHUNK,), jnp.float32),
        pltpu.VMEM((CHUNK,), jnp.float32),
        pltpu.VMEM((LANES,), jnp.float32),
        pltpu.VMEM((LANES,), jnp.float32),
        pltpu.SemaphoreType.DMA,
        pltpu.SemaphoreType.DMA,
    ],
)
def _minmax_kernel(x_hbm, min_hbm, max_hbm, in0, in1, mn_buf, mx_buf, si0, si1):
    wid = _wid()
    r0 = wid * ROWS_W
    bufs = ((in0, si0), (in1, si1))

    _rows_in(x_hbm, r0, in0, si0)
    _rows_in(x_hbm, r0 + RCHUNK, in1, si1)

    def pair(p, carry):
        for b, (inb, si) in enumerate(bufs):
            ci = p * 2 + b
            rr = r0 + ci * RCHUNK
            _rows_in_wait(x_hbm, rr, inb, si)

            def vred(i, acc):
                off = i * 128
                new = []
                for k in range(8):
                    v = inb[pl.ds(off + k * LANES, LANES)]
                    new.append(jnp.minimum(acc[k], v))
                for k in range(8):
                    v = inb[pl.ds(off + k * LANES, LANES)]
                    new.append(jnp.maximum(acc[8 + k], v))
                return tuple(new)

            carry = lax.fori_loop(0, CHUNK // 128, vred, carry)

            @pl.when(ci + 2 < NCHUNK)
            def _():
                _rows_in(x_hbm, rr + 2 * RCHUNK, inb, si)
        return carry

    init = tuple(
        jnp.full((LANES,), jnp.inf if k < 8 else -jnp.inf, jnp.float32)
        for k in range(16)
    )
    acc = lax.fori_loop(0, NCHUNK // 2, pair, init)
    mn = acc[0]
    mx = acc[8]
    for k in range(1, 8):
        mn = jnp.minimum(mn, acc[k])
        mx = jnp.maximum(mx, acc[8 + k])
    mn_buf[...] = mn
    mx_buf[...] = mx
    pltpu.sync_copy(mn_buf, min_hbm.at[pl.ds(wid * LANES, LANES)])
    pltpu.sync_copy(mx_buf, max_hbm.at[pl.ds(wid * LANES, LANES)])


def _ce(a, b):
    return jnp.maximum(a, b), jnp.minimum(a, b)


def _windows(buf, sv, iv):
    idx8 = lax.iota(jnp.int32, LANES) * 8
    zeroi = jnp.zeros((LANES,), jnp.int32)

    @plsc.parallel_loop(0, NWIN, 1, unroll=2)
    def window(w):
        base = w * 128
        idxs = [idx8 + (base + j) for j in range(8)]
        xs = [plsc.load_gather(buf, [idxs[j]]) for j in range(8)]
        ms = []
        for j in range(8):
            t = xs[j] * iv
            u = t + MAGICF
            q0 = plsc.bitcast(u, jnp.int32) - MAGICI
            ms.append(jnp.abs(jnp.minimum(jnp.maximum(q0, -128), 127)))
        a0, a1 = _ce(ms[0], ms[1])
        a2, a3 = _ce(ms[2], ms[3])
        a0, a2 = _ce(a0, a2)
        a1, a3 = _ce(a1, a3)
        a1, a2 = _ce(a1, a2)
        b0, b1 = _ce(ms[4], ms[5])
        b2, b3 = _ce(ms[6], ms[7])
        b0, b2 = _ce(b0, b2)
        b1, b3 = _ce(b1, b3)
        b1, b2 = _ce(b1, b2)
        thr = jnp.maximum(
            jnp.maximum(b3, jnp.minimum(a0, b2)),
            jnp.maximum(jnp.minimum(a1, b1),
                        jnp.maximum(jnp.minimum(a2, b0), a3)))
        for j in range(8):
            low = ms[j] & 15
            drop = jnp.where(ms[j] >= thr, zeroi, low)
            k2 = ms[j] - drop
            fs = k2.astype(jnp.float32) * sv
            ob = plsc.bitcast(
                plsc.bitcast(fs, jnp.int32)
                | (plsc.bitcast(xs[j], jnp.int32) & SIGNBIT),
                jnp.float32)
            plsc.store_scatter(buf, [idxs[j]], ob)


@functools.partial(
    pl.kernel,
    out_type=jax.ShapeDtypeStruct((R, C), jnp.float32),
    mesh=_mesh,
    compiler_params=_params,
    scratch_types=[
        pltpu.VMEM((CHUNK,), jnp.float32),
        pltpu.VMEM((CHUNK,), jnp.float32),
        pltpu.VMEM((NW * LANES,), jnp.float32),
        pltpu.VMEM((NW * LANES,), jnp.float32),
        pltpu.SemaphoreType.DMA,
        pltpu.SemaphoreType.DMA,
        pltpu.SemaphoreType.DMA,
        pltpu.SemaphoreType.DMA,
    ],
)
def _quant_kernel(x_hbm, min_hbm, max_hbm, out_hbm,
                  buf0, buf1, mnb, mxb, si0, si1, so0, so1):
    wid = _wid()
    r0 = wid * ROWS_W
    bufs = ((buf0, si0, so0), (buf1, si1, so1))

    _rows_in(x_hbm, r0, buf0, si0)
    _rows_in(x_hbm, r0 + RCHUNK, buf1, si1)

    pltpu.sync_copy(min_hbm, mnb)
    pltpu.sync_copy(max_hbm, mxb)
    mnv = mnb[pl.ds(0, LANES)]
    mxv = mxb[pl.ds(0, LANES)]
    for w in range(1, NW):
        mnv = jnp.minimum(mnv, mnb[pl.ds(w * LANES, LANES)])
        mxv = jnp.maximum(mxv, mxb[pl.ds(w * LANES, LANES)])
    rmin = jnp.full((LANES,), jnp.min(mnv), jnp.float32)
    rmax = jnp.full((LANES,), jnp.max(mxv), jnp.float32)
    rmin = jnp.minimum(rmin, 0.0)
    rmax = jnp.maximum(rmax, 0.0)
    sv = jnp.maximum((rmax - rmin) / 255.0, 1e-8)
    iv = jnp.float32(1.0) / sv

    def pair(p, _):
        for b, (buf, si, so) in enumerate(bufs):
            ci = p * 2 + b
            rr = r0 + ci * RCHUNK
            obuf, osi, oso = bufs[1 - b]
            _rows_in_wait(x_hbm, rr, buf, si)
            _windows(buf, sv, iv)
            _rows_out(out_hbm, rr, buf, so)

            @pl.when((ci >= 1) & (ci + 1 < NCHUNK))
            def _():
                _rows_out_wait(out_hbm, r0, obuf, oso)
                _rows_in(x_hbm, rr + RCHUNK, obuf, osi)
        return 0

    lax.fori_loop(0, NCHUNK // 2, pair, 0)
    _rows_out_wait(out_hbm, r0, buf0, so0)
    _rows_out_wait(out_hbm, r0, buf1, so1)


def kernel(x):
    x2 = x.reshape(R, C)
    mn, mx = _minmax_kernel(x2)
    out = _quant_kernel(x2, mn, mx)
    return out.reshape(x.shape)


# speedup vs baseline: 8.3958x; 1.2197x over previous
import functools

import jax
import jax.numpy as jnp
import numpy as np
from jax import lax
from jax.experimental import pallas as pl
from jax.experimental.pallas import tpu as pltpu
from jax.experimental.pallas import tpu_sc as plsc

NC = 2
NS = 16
LANES = 16
NW = NC * NS

R = 8192
C = 4096
ROWS_W = R // NW
RCHUNK = 8
NCHUNK = ROWS_W // RCHUNK
BRCHUNK = 4
BNCHUNK = ROWS_W // BRCHUNK
BCHUNK = BRCHUNK * C
CHUNK = RCHUNK * C
NWIN = BCHUNK // 128

MAGICF = np.float32(12582912.0)
MAGICI = np.int32(0x4B400000)
SIGNBIT = np.int32(np.uint32(0x80000000).view(np.int32))

_mesh = plsc.VectorSubcoreMesh(core_axis_name="c", subcore_axis_name="s")
_params = pltpu.CompilerParams(needs_layout_passes=False)


def _wid():
    return lax.axis_index("s") * NC + lax.axis_index("c")


def _rows_in(x_hbm, rr, buf, sem, nrows=RCHUNK):
    for s in range(nrows):
        pltpu.async_copy(x_hbm.at[rr + s, :], buf.at[pl.ds(s * C, C)], sem)


def _rows_in_wait(x_hbm, rr, buf, sem, nrows=RCHUNK):
    for s in range(nrows):
        pltpu.make_async_copy(x_hbm.at[rr + s, :], buf.at[pl.ds(s * C, C)],
                              sem).wait()


def _rows_out(out_hbm, rr, buf, sem, nrows=RCHUNK):
    for s in range(nrows):
        pltpu.async_copy(buf.at[pl.ds(s * C, C)], out_hbm.at[rr + s, :], sem)


def _rows_out_wait(out_hbm, rr, buf, sem, nrows=RCHUNK):
    for s in range(nrows):
        pltpu.make_async_copy(buf.at[pl.ds(s * C, C)], out_hbm.at[rr + s, :],
                              sem).wait()


@functools.partial(
    pl.kernel,
    out_type=(
        jax.ShapeDtypeStruct((NW * LANES,), jnp.float32),
        jax.ShapeDtypeStruct((NW * LANES,), jnp.float32),
    ),
    mesh=_mesh,
    compiler_params=_params,
    scratch_types=[
        pltpu.VMEM((CHUNK,), jnp.float32),
        pltpu.VMEM((CHUNK,), jnp.float32),
        pltpu.VMEM((LANES,), jnp.float32),
        pltpu.VMEM((LANES,), jnp.float32),
        pltpu.SemaphoreType.DMA,
        pltpu.SemaphoreType.DMA,
    ],
)
def _minmax_kernel(x_hbm, min_hbm, max_hbm, in0, in1, mn_buf, mx_buf, si0, si1):
    wid = _wid()
    r0 = wid * ROWS_W
    bufs = ((in0, si0), (in1, si1))

    _rows_in(x_hbm, r0, in0, si0)
    _rows_in(x_hbm, r0 + RCHUNK, in1, si1)

    def pair(p, carry):
        for b, (inb, si) in enumerate(bufs):
            ci = p * 2 + b
            rr = r0 + ci * RCHUNK
            _rows_in_wait(x_hbm, rr, inb, si)

            def vred(i, acc):
                off = i * 128
                new = []
                for k in range(8):
                    v = inb[pl.ds(off + k * LANES, LANES)]
                    new.append(jnp.minimum(acc[k], v))
                for k in range(8):
                    v = inb[pl.ds(off + k * LANES, LANES)]
                    new.append(jnp.maximum(acc[8 + k], v))
                return tuple(new)

            carry = lax.fori_loop(0, CHUNK // 128, vred, carry)

            @pl.when(ci + 2 < NCHUNK)
            def _():
                _rows_in(x_hbm, rr + 2 * RCHUNK, inb, si)
        return carry

    init = tuple(
        jnp.full((LANES,), jnp.inf if k < 8 else -jnp.inf, jnp.float32)
        for k in range(16)
    )
    acc = lax.fori_loop(0, NCHUNK // 2, pair, init)
    mn = acc[0]
    mx = acc[8]
    for k in range(1, 8):
        mn = jnp.minimum(mn, acc[k])
        mx = jnp.maximum(mx, acc[8 + k])
    mn_buf[...] = mn
    mx_buf[...] = mx
    pltpu.sync_copy(mn_buf, min_hbm.at[pl.ds(wid * LANES, LANES)])
    pltpu.sync_copy(mx_buf, max_hbm.at[pl.ds(wid * LANES, LANES)])


def _ce(a, b):
    return jnp.maximum(a, b), jnp.minimum(a, b)


def _windows(inb, outb, sv, iv):
    idx8 = lax.iota(jnp.int32, LANES) * 8
    zeroi = jnp.zeros((LANES,), jnp.int32)

    @plsc.parallel_loop(0, NWIN, 1, unroll=2)
    def window(w):
        base = w * 128
        idxs = [idx8 + (base + j) for j in range(8)]
        xs = [plsc.load_gather(inb, [idxs[j]]) for j in range(8)]
        ms = []
        for j in range(8):
            t = xs[j] * iv
            u = t + MAGICF
            q0 = plsc.bitcast(u, jnp.int32) - MAGICI
            ms.append(jnp.abs(jnp.minimum(jnp.maximum(q0, -128), 127)))
        a0, a1 = _ce(ms[0], ms[1])
        a2, a3 = _ce(ms[2], ms[3])
        a0, a2 = _ce(a0, a2)
        a1, a3 = _ce(a1, a3)
        a1, a2 = _ce(a1, a2)
        b0, b1 = _ce(ms[4], ms[5])
        b2, b3 = _ce(ms[6], ms[7])
        b0, b2 = _ce(b0, b2)
        b1, b3 = _ce(b1, b3)
        b1, b2 = _ce(b1, b2)
        thr = jnp.maximum(
            jnp.maximum(b3, jnp.minimum(a0, b2)),
            jnp.maximum(jnp.minimum(a1, b1),
                        jnp.maximum(jnp.minimum(a2, b0), a3)))
        for j in range(8):
            low = ms[j] & 15
            drop = jnp.where(ms[j] >= thr, zeroi, low)
            k2 = ms[j] - drop
            fs = k2.astype(jnp.float32) * sv
            ob = plsc.bitcast(
                plsc.bitcast(fs, jnp.int32)
                | (plsc.bitcast(xs[j], jnp.int32) & SIGNBIT),
                jnp.float32)
            plsc.store_scatter(outb, [idxs[j]], ob)


@functools.partial(
    pl.kernel,
    out_type=jax.ShapeDtypeStruct((R, C), jnp.float32),
    mesh=_mesh,
    compiler_params=_params,
    scratch_types=[
        pltpu.VMEM((BCHUNK,), jnp.float32),
        pltpu.VMEM((BCHUNK,), jnp.float32),
        pltpu.VMEM((BCHUNK,), jnp.float32),
        pltpu.VMEM((BCHUNK,), jnp.float32),
        pltpu.VMEM((NW * LANES,), jnp.float32),
        pltpu.VMEM((NW * LANES,), jnp.float32),
        pltpu.SemaphoreType.DMA,
        pltpu.SemaphoreType.DMA,
        pltpu.SemaphoreType.DMA,
        pltpu.SemaphoreType.DMA,
    ],
)
def _quant_kernel(x_hbm, min_hbm, max_hbm, out_hbm,
                  in0, in1, out0, out1, mnb, mxb, si0, si1, so0, so1):
    wid = _wid()
    r0 = wid * ROWS_W
    bufs = ((in0, out0, si0, so0), (in1, out1, si1, so1))

    _rows_in(x_hbm, r0, in0, si0, BRCHUNK)
    _rows_in(x_hbm, r0 + BRCHUNK, in1, si1, BRCHUNK)

    pltpu.sync_copy(min_hbm, mnb)
    pltpu.sync_copy(max_hbm, mxb)
    mnv = mnb[pl.ds(0, LANES)]
    mxv = mxb[pl.ds(0, LANES)]
    for w in range(1, NW):
        mnv = jnp.minimum(mnv, mnb[pl.ds(w * LANES, LANES)])
        mxv = jnp.maximum(mxv, mxb[pl.ds(w * LANES, LANES)])
    rmin = jnp.full((LANES,), jnp.min(mnv), jnp.float32)
    rmax = jnp.full((LANES,), jnp.max(mxv), jnp.float32)
    rmin = jnp.minimum(rmin, 0.0)
    rmax = jnp.maximum(rmax, 0.0)
    sv = jnp.maximum((rmax - rmin) / 255.0, 1e-8)
    iv = jnp.float32(1.0) / sv

    def pair(p, _):
        for b, (inb, outb, si, so) in enumerate(bufs):
            ci = p * 2 + b
            rr = r0 + ci * BRCHUNK
            _rows_in_wait(x_hbm, rr, inb, si, BRCHUNK)

            @pl.when(ci >= 2)
            def _():
                _rows_out_wait(out_hbm, r0, outb, so, BRCHUNK)

            _windows(inb, outb, sv, iv)
            _rows_out(out_hbm, rr, outb, so, BRCHUNK)

            @pl.when(ci + 2 < BNCHUNK)
            def _():
                _rows_in(x_hbm, rr + 2 * BRCHUNK, inb, si, BRCHUNK)
        return 0

    lax.fori_loop(0, BNCHUNK // 2, pair, 0)
    _rows_out_wait(out_hbm, r0, out0, so0, BRCHUNK)
    _rows_out_wait(out_hbm, r0, out1, so1, BRCHUNK)


def kernel(x):
    x2 = x.reshape(R, C)
    mn, mx = _minmax_kernel(x2)
    out = _quant_kernel(x2, mn, mx)
    return out.reshape(x.shape)
